# baseline (speedup 1.0000x reference)
import functools
import math

import numpy as np
import jax
import jax.numpy as jnp
from jax import lax
from jax.experimental import pallas as pl
from jax.experimental.pallas import tpu as pltpu
from jax.experimental.pallas import tpu_sc as plsc

B = 16384
NW = 32
ROWS = B // NW
GROUPS = ROWS // 16
NCHUNK = 4
CHUNK = ROWS // NCHUNK
INV = 1.0 / math.sqrt(1.0 + 1e-5)
HMAX = 392

_F7, _F24 = 5, 9


def _pack(v: np.ndarray) -> np.ndarray:
    return (v % 12) | ((v % 7) << _F7) | ((v % 24) << _F24)


_HAP = _pack(256 * np.arange(HMAX, dtype=np.int64)).astype(np.int32)
_HBP = _pack(np.arange(256, dtype=np.int64)).astype(np.int32)


def _sincos_ext(m: int, n: int, phase: int) -> np.ndarray:
    k = np.arange(n, dtype=np.float64) + phase
    ang = 2.0 * np.pi * k / m
    return np.stack([np.sin(ang), np.cos(ang)], axis=1).reshape(-1)


_T12 = _sincos_ext(12, 23, 11)
_T7 = _sincos_ext(7, 13, 0)
_T24 = _sincos_ext(24, 47, 0)
_OFF12, _OFF7, _OFF24 = 0, _T12.size, _T12.size + _T7.size
_TTAB = np.concatenate([_T12, _T7, _T24]).astype(np.float32)
_TTAB = np.pad(_TTAB, (0, (-_TTAB.size) % 8))
TTN = _TTAB.size

_mesh = plsc.VectorSubcoreMesh(core_axis_name="c", subcore_axis_name="s")


@functools.partial(
    pl.kernel,
    mesh=_mesh,
    compiler_params=pltpu.CompilerParams(
        needs_layout_passes=False, use_tc_tiling_on_sc=False),
    out_type=jax.ShapeDtypeStruct((B * 12,), jnp.float32),
    scratch_types=[
        pltpu.VMEM((ROWS * 6,), jnp.int32),
        [[pltpu.VMEM((NCHUNK, CHUNK), jnp.int32) for _ in range(2)]
         for _ in range(3)],
        pltpu.VMEM((HMAX,), jnp.int32),
        pltpu.VMEM((256,), jnp.int32),
        pltpu.VMEM((TTN,), jnp.float32),
        pltpu.VMEM((16,), jnp.float32),
        pltpu.VMEM((16,), jnp.float32),
        [[pltpu.VMEM((ROWS,), jnp.float32) for _ in range(2)]
         for _ in range(3)],
        pltpu.VMEM((ROWS * 12,), jnp.float32),
        pltpu.SemaphoreType.DMA,
    ],
)
def _encode(x_hbm, hap_hbm, hbp_hbm, ttab_hbm, e3_hbm, e4_hbm, e5_hbm,
            g_hbm, b_hbm, out_hbm,
            x_v, i_v, hap_v, hbp_v, t_v, g_v, b_v, e_v, o_v, sem):
    wid = lax.axis_index("s") * 2 + lax.axis_index("c")
    base = wid * ROWS

    pltpu.sync_copy(x_hbm.at[pl.ds(base * 6, ROWS * 6)], x_v)
    pltpu.sync_copy(hap_hbm, hap_v)
    pltpu.sync_copy(hbp_hbm, hbp_v)
    pltpu.sync_copy(ttab_hbm, t_v)
    pltpu.sync_copy(g_hbm, g_v)
    pltpu.sync_copy(b_hbm, b_v)

    i16 = lax.iota(jnp.int32, 16)

    for g in range(GROUPS):
        r, o = g // 8, (g % 8) * 16
        xb = 96 * g + 6 * i16
        for t in range(3):
            w = 2 * plsc.load_gather(x_v, [xb + 3 + t])
            i_v[t][0][r, pl.ds(o, 16)] = w
            i_v[t][1][r, pl.ds(o, 16)] = w + 1
    copies = []
    for t, tab_hbm in enumerate((e3_hbm, e4_hbm, e5_hbm)):
        for col in range(2):
            for c in range(NCHUNK):
                copies.append(pltpu.async_copy(
                    tab_hbm.at[i_v[t][col].at[c]],
                    e_v[t][col].at[pl.ds(c * CHUNK, CHUNK)], sem))

    gvec = g_v[...]
    bvec = b_v[...]
    scale = [gvec[j] * INV for j in range(12)]
    bias = [bvec[j] for j in range(12)]

    def emit(col, t, ob):
        v = jnp.maximum(t * scale[col] + bias[col], 0.0)
        plsc.store_scatter(o_v, [ob + col], v)

    for g in range(GROUPS):
        ob = 192 * g + 12 * i16
        for j in range(3):
            xv = plsc.load_gather(x_v, [96 * g + 6 * i16 + j])
            pa = plsc.load_gather(hap_v, [lax.shift_right_logical(xv, 8)])
            pb = plsc.load_gather(hbp_v, [lax.bitwise_and(xv, 255)])
            s = pa + pb
            if j == 0:
                t = _OFF12 + 2 * lax.bitwise_and(s, 31)
            elif j == 1:
                t = _OFF7 + 2 * lax.bitwise_and(
                    lax.shift_right_logical(s, _F7), 15)
            else:
                t = _OFF24 + 2 * lax.shift_right_logical(s, _F24)
            emit(2 * j, plsc.load_gather(t_v, [t]), ob)
            emit(2 * j + 1, plsc.load_gather(t_v, [t + 1]), ob)

    for cp in copies:
        cp.wait()

    for g in range(GROUPS):
        ob = 192 * g + 12 * i16
        for t in range(3):
            for col in range(2):
                v = e_v[t][col][pl.ds(16 * g, 16)]
                emit(6 + 2 * t + col, v, ob)

    pltpu.sync_copy(o_v, out_hbm.at[pl.ds(base * 12, ROWS * 12)])


def kernel(x, emb3, emb4, emb5, gamma, beta):
    xi = x.astype(jnp.int32).reshape(-1)
    g16 = jnp.concatenate([gamma, jnp.zeros((4,), jnp.float32)])
    b16 = jnp.concatenate([beta, jnp.zeros((4,), jnp.float32)])
    out = _encode(xi, jnp.asarray(_HAP), jnp.asarray(_HBP),
                  jnp.asarray(_TTAB), emb3.reshape(-1), emb4.reshape(-1),
                  emb5.reshape(-1), g16, b16)
    return out.reshape(B, 12)


# speedup vs baseline: 1.0002x; 1.0002x over previous
import functools
import math

import numpy as np
import jax
import jax.numpy as jnp
from jax import lax
from jax.experimental import pallas as pl
from jax.experimental.pallas import tpu as pltpu
from jax.experimental.pallas import tpu_sc as plsc

B = 16384
NW = 32
ROWS = B // NW
GROUPS = ROWS // 16
NCHUNK = 4
CHUNK = ROWS // NCHUNK
INV = 1.0 / math.sqrt(1.0 + 1e-5)
HMAX = 392

_F7, _F24 = 5, 9


def _pack(v: np.ndarray) -> np.ndarray:
    return (v % 12) | ((v % 7) << _F7) | ((v % 24) << _F24)


_HAP = _pack(256 * np.arange(HMAX, dtype=np.int64)).astype(np.int32)
_HBP = _pack(np.arange(256, dtype=np.int64)).astype(np.int32)


def _sincos_ext(m: int, n: int, phase: int) -> np.ndarray:
    k = np.arange(n, dtype=np.float64) + phase
    ang = 2.0 * np.pi * k / m
    return np.stack([np.sin(ang), np.cos(ang)], axis=1).reshape(-1)


_T12 = _sincos_ext(12, 23, 11)
_T7 = _sincos_ext(7, 13, 0)
_T24 = _sincos_ext(24, 47, 0)
_OFF12, _OFF7, _OFF24 = 0, _T12.size, _T12.size + _T7.size
_TTAB = np.concatenate([_T12, _T7, _T24]).astype(np.float32)
_TTAB = np.pad(_TTAB, (0, (-_TTAB.size) % 8))
TTN = _TTAB.size

_mesh = plsc.VectorSubcoreMesh(core_axis_name="c", subcore_axis_name="s")


@functools.partial(
    pl.kernel,
    mesh=_mesh,
    compiler_params=pltpu.CompilerParams(
        needs_layout_passes=False, use_tc_tiling_on_sc=False,
        skip_device_barrier=True, disable_bounds_checks=True,
        disable_semaphore_checks=True),
    out_type=jax.ShapeDtypeStruct((B * 12,), jnp.float32),
    scratch_types=[
        pltpu.VMEM((ROWS * 6,), jnp.int32),
        [[pltpu.VMEM((NCHUNK, CHUNK), jnp.int32) for _ in range(2)]
         for _ in range(3)],
        pltpu.VMEM((HMAX,), jnp.int32),
        pltpu.VMEM((256,), jnp.int32),
        pltpu.VMEM((TTN,), jnp.float32),
        pltpu.VMEM((16,), jnp.float32),
        pltpu.VMEM((16,), jnp.float32),
        [[pltpu.VMEM((ROWS,), jnp.float32) for _ in range(2)]
         for _ in range(3)],
        pltpu.VMEM((ROWS * 12,), jnp.float32),
        pltpu.SemaphoreType.DMA,
    ],
)
def _encode(x_hbm, hap_hbm, hbp_hbm, ttab_hbm, e3_hbm, e4_hbm, e5_hbm,
            g_hbm, b_hbm, out_hbm,
            x_v, i_v, hap_v, hbp_v, t_v, g_v, b_v, e_v, o_v, sem):
    wid = lax.axis_index("s") * 2 + lax.axis_index("c")
    base = wid * ROWS

    pltpu.sync_copy(x_hbm.at[pl.ds(base * 6, ROWS * 6)], x_v)
    pltpu.sync_copy(hap_hbm, hap_v)
    pltpu.sync_copy(hbp_hbm, hbp_v)
    pltpu.sync_copy(ttab_hbm, t_v)
    pltpu.sync_copy(g_hbm, g_v)
    pltpu.sync_copy(b_hbm, b_v)

    i16 = lax.iota(jnp.int32, 16)

    for g in range(GROUPS):
        r, o = g // 8, (g % 8) * 16
        xb = 96 * g + 6 * i16
        for t in range(3):
            w = 2 * plsc.load_gather(x_v, [xb + 3 + t])
            i_v[t][0][r, pl.ds(o, 16)] = w
            i_v[t][1][r, pl.ds(o, 16)] = w + 1
    copies = []
    for t, tab_hbm in enumerate((e3_hbm, e4_hbm, e5_hbm)):
        for col in range(2):
            for c in range(NCHUNK):
                copies.append(pltpu.async_copy(
                    tab_hbm.at[i_v[t][col].at[c]],
                    e_v[t][col].at[pl.ds(c * CHUNK, CHUNK)], sem))

    gvec = g_v[...]
    bvec = b_v[...]
    scale = [gvec[j] * INV for j in range(12)]
    bias = [bvec[j] for j in range(12)]

    def emit(col, t, ob):
        v = jnp.maximum(t * scale[col] + bias[col], 0.0)
        plsc.store_scatter(o_v, [ob + col], v)

    for g in range(GROUPS):
        ob = 192 * g + 12 * i16
        for j in range(3):
            xv = plsc.load_gather(x_v, [96 * g + 6 * i16 + j])
            pa = plsc.load_gather(hap_v, [lax.shift_right_logical(xv, 8)])
            pb = plsc.load_gather(hbp_v, [lax.bitwise_and(xv, 255)])
            s = pa + pb
            if j == 0:
                t = _OFF12 + 2 * lax.bitwise_and(s, 31)
            elif j == 1:
                t = _OFF7 + 2 * lax.bitwise_and(
                    lax.shift_right_logical(s, _F7), 15)
            else:
                t = _OFF24 + 2 * lax.shift_right_logical(s, _F24)
            emit(2 * j, plsc.load_gather(t_v, [t]), ob)
            emit(2 * j + 1, plsc.load_gather(t_v, [t + 1]), ob)

    for cp in copies:
        cp.wait()

    for g in range(GROUPS):
        ob = 192 * g + 12 * i16
        for t in range(3):
            for col in range(2):
                v = e_v[t][col][pl.ds(16 * g, 16)]
                emit(6 + 2 * t + col, v, ob)

    pltpu.sync_copy(o_v, out_hbm.at[pl.ds(base * 12, ROWS * 12)])


def kernel(x, emb3, emb4, emb5, gamma, beta):
    xi = x.astype(jnp.int32).reshape(-1)
    g16 = jnp.concatenate([gamma, jnp.zeros((4,), jnp.float32)])
    b16 = jnp.concatenate([beta, jnp.zeros((4,), jnp.float32)])
    out = _encode(xi, jnp.asarray(_HAP), jnp.asarray(_HBP),
                  jnp.asarray(_TTAB), emb3.reshape(-1), emb4.reshape(-1),
                  emb5.reshape(-1), g16, b16)
    return out.reshape(B, 12)


# speedup vs baseline: 3.7976x; 3.7968x over previous
import functools
import math

import numpy as np
import jax
import jax.numpy as jnp
from jax import lax
from jax.experimental import pallas as pl
from jax.experimental.pallas import tpu as pltpu
from jax.experimental.pallas import tpu_sc as plsc

B = 16384
NW = 32
ROWS = B // NW
GROUPS = ROWS // 16
NCHUNK = 4
CHUNK = ROWS // NCHUNK
INV = 1.0 / math.sqrt(1.0 + 1e-5)
HMAX = 392
V = 100001

_F7, _F24 = 5, 9


def _pack(v: np.ndarray) -> np.ndarray:
    return (v % 12) | ((v % 7) << _F7) | ((v % 24) << _F24)


_HAP = _pack(256 * np.arange(HMAX, dtype=np.int64)).astype(np.int32)
_HBP = _pack(np.arange(256, dtype=np.int64)).astype(np.int32)


def _sincos_ext(m: int, n: int, phase: int) -> np.ndarray:
    k = np.arange(n, dtype=np.float64) + phase
    ang = 2.0 * np.pi * k / m
    return np.stack([np.sin(ang), np.cos(ang)], axis=1).reshape(-1)


_T12 = _sincos_ext(12, 23, 11)
_T7 = _sincos_ext(7, 13, 0)
_T24 = _sincos_ext(24, 47, 0)
_OFF12, _OFF7, _OFF24 = 0, _T12.size, _T12.size + _T7.size
_TTAB = np.concatenate([_T12, _T7, _T24]).astype(np.float32)
_TTAB = np.pad(_TTAB, (0, (-_TTAB.size) % 8))
TTN = _TTAB.size

_mesh = plsc.VectorSubcoreMesh(core_axis_name="c", subcore_axis_name="s")


@functools.partial(
    pl.kernel,
    mesh=_mesh,
    compiler_params=pltpu.CompilerParams(
        needs_layout_passes=False, use_tc_tiling_on_sc=False),
    out_type=jax.ShapeDtypeStruct((B * 12,), jnp.float32),
    scratch_types=[
        [pltpu.VMEM((ROWS,), jnp.int32) for _ in range(6)],
        [pltpu.VMEM((ROWS,), jnp.int32) for _ in range(3)],
        pltpu.VMEM((HMAX,), jnp.int32),
        pltpu.VMEM((256,), jnp.int32),
        pltpu.VMEM((TTN,), jnp.float32),
        pltpu.VMEM((16,), jnp.float32),
        pltpu.VMEM((16,), jnp.float32),
        [[pltpu.VMEM((ROWS,), jnp.float32) for _ in range(2)]
         for _ in range(3)],
        pltpu.VMEM((ROWS * 12,), jnp.float32),
        pltpu.SemaphoreType.DMA,
    ],
)
def _encode(xt_hbm, hap_hbm, hbp_hbm, ttab_hbm, f3_hbm, f4_hbm, f5_hbm,
            g_hbm, b_hbm, out_hbm,
            c_v, ib_v, hap_v, hbp_v, t_v, g_v, b_v, e_v, o_v, sem):
    wid = lax.axis_index("s") * 2 + lax.axis_index("c")
    base = wid * ROWS
    i16 = lax.iota(jnp.int32, 16)
    tabs = (f3_hbm, f4_hbm, f5_hbm)

    for j in range(6):
        pltpu.sync_copy(xt_hbm.at[pl.ds(j * B + base, ROWS)], c_v[j])
    pltpu.sync_copy(hap_hbm, hap_v)
    pltpu.sync_copy(hbp_hbm, hbp_v)
    pltpu.sync_copy(ttab_hbm, t_v)
    pltpu.sync_copy(g_hbm, g_v.at[pl.ds(0, 12)])
    pltpu.sync_copy(b_hbm, b_v.at[pl.ds(0, 12)])

    for t in range(3):
        for g in range(GROUPS):
            ib_v[t][pl.ds(16 * g, 16)] = c_v[3 + t][pl.ds(16 * g, 16)] + V

    copies = []
    for t in range(3):
        for c in range(NCHUNK):
            sl = pl.ds(c * CHUNK, CHUNK)
            copies.append(pltpu.async_copy(
                tabs[t].at[c_v[3 + t].at[sl]], e_v[t][0].at[sl], sem))
            copies.append(pltpu.async_copy(
                tabs[t].at[ib_v[t].at[sl]], e_v[t][1].at[sl], sem))

    gvec = g_v[...]
    bvec = b_v[...]
    scale = [gvec[j] * INV for j in range(12)]
    bias = [bvec[j] for j in range(12)]

    def emit(col, t, ob):
        v = jnp.maximum(t * scale[col] + bias[col], 0.0)
        plsc.store_scatter(o_v, [ob + col], v)

    for g in range(GROUPS):
        ob = 192 * g + 12 * i16
        for j in range(3):
            xv = c_v[j][pl.ds(16 * g, 16)]
            pa = plsc.load_gather(hap_v, [lax.shift_right_logical(xv, 8)])
            pb = plsc.load_gather(hbp_v, [lax.bitwise_and(xv, 255)])
            s = pa + pb
            if j == 0:
                t = _OFF12 + 2 * lax.bitwise_and(s, 31)
            elif j == 1:
                t = _OFF7 + 2 * lax.bitwise_and(
                    lax.shift_right_logical(s, _F7), 15)
            else:
                t = _OFF24 + 2 * lax.shift_right_logical(s, _F24)
            emit(2 * j, plsc.load_gather(t_v, [t]), ob)
            emit(2 * j + 1, plsc.load_gather(t_v, [t + 1]), ob)

    for cp in copies:
        cp.wait()

    for g in range(GROUPS):
        ob = 192 * g + 12 * i16
        for t in range(3):
            for col in range(2):
                emit(6 + 2 * t + col, e_v[t][col][pl.ds(16 * g, 16)], ob)

    pltpu.sync_copy(o_v, out_hbm.at[pl.ds(base * 12, ROWS * 12)])


def kernel(x, emb3, emb4, emb5, gamma, beta):
    xt = x.astype(jnp.int32).T.reshape(-1)
    out = _encode(xt, jnp.asarray(_HAP), jnp.asarray(_HBP),
                  jnp.asarray(_TTAB), emb3.T.reshape(-1), emb4.T.reshape(-1),
                  emb5.T.reshape(-1), gamma, beta)
    return out.reshape(B, 12)


# speedup vs baseline: 4.0957x; 1.0785x over previous
import functools
import math

import numpy as np
import jax
import jax.numpy as jnp
from jax import lax
from jax.experimental import pallas as pl
from jax.experimental.pallas import tpu as pltpu
from jax.experimental.pallas import tpu_sc as plsc

B = 16384
NW = 32
ROWS = B // NW
GROUPS = ROWS // 16
NCHUNK = 4
CHUNK = ROWS // NCHUNK
INV = 1.0 / math.sqrt(1.0 + 1e-5)
HMAX = 392
V = 100001

_F7, _F24 = 5, 9


def _pack(v: np.ndarray) -> np.ndarray:
    return (v % 12) | ((v % 7) << _F7) | ((v % 24) << _F24)


_HAP = _pack(256 * np.arange(HMAX, dtype=np.int64)).astype(np.int32)
_HBP = _pack(np.arange(256, dtype=np.int64)).astype(np.int32)


def _sincos_ext(m: int, n: int, phase: int) -> np.ndarray:
    k = np.arange(n, dtype=np.float64) + phase
    ang = 2.0 * np.pi * k / m
    return np.stack([np.sin(ang), np.cos(ang)], axis=1).reshape(-1)


_T12 = _sincos_ext(12, 23, 11)
_T7 = _sincos_ext(7, 13, 0)
_T24 = _sincos_ext(24, 47, 0)
_OFF12, _OFF7, _OFF24 = 0, _T12.size, _T12.size + _T7.size
_TTAB = np.concatenate([_T12, _T7, _T24]).astype(np.float32)
_TTAB = np.pad(_TTAB, (0, (-_TTAB.size) % 8))
TTN = _TTAB.size

_mesh = plsc.VectorSubcoreMesh(core_axis_name="c", subcore_axis_name="s")


@functools.partial(
    pl.kernel,
    mesh=_mesh,
    compiler_params=pltpu.CompilerParams(
        needs_layout_passes=False, use_tc_tiling_on_sc=False),
    out_type=jax.ShapeDtypeStruct((B * 12,), jnp.float32),
    scratch_types=[
        [pltpu.VMEM((ROWS,), jnp.int32) for _ in range(6)],
        [pltpu.VMEM((ROWS,), jnp.int32) for _ in range(3)],
        pltpu.VMEM((HMAX,), jnp.int32),
        pltpu.VMEM((256,), jnp.int32),
        pltpu.VMEM((TTN,), jnp.float32),
        pltpu.VMEM((16,), jnp.float32),
        pltpu.VMEM((16,), jnp.float32),
        [[pltpu.VMEM((ROWS,), jnp.float32) for _ in range(2)]
         for _ in range(3)],
        pltpu.VMEM((ROWS * 12,), jnp.float32),
        pltpu.SemaphoreType.DMA,
        pltpu.SemaphoreType.DMA,
        pltpu.SemaphoreType.DMA,
    ],
)
def _encode(xt_hbm, hap_hbm, hbp_hbm, ttab_hbm, f3_hbm, f4_hbm, f5_hbm,
            g_hbm, b_hbm, out_hbm,
            c_v, ib_v, hap_v, hbp_v, t_v, g_v, b_v, e_v, o_v,
            sem, sem_i, sem_a):
    wid = lax.axis_index("s") * 2 + lax.axis_index("c")
    base = wid * ROWS
    i16 = lax.iota(jnp.int32, 16)
    tabs = (f3_hbm, f4_hbm, f5_hbm)

    idx_cps = [pltpu.async_copy(
        xt_hbm.at[pl.ds((3 + t) * B + base, ROWS)], c_v[3 + t], sem_i)
        for t in range(3)]
    aux_cps = [pltpu.async_copy(
        xt_hbm.at[pl.ds(j * B + base, ROWS)], c_v[j], sem_a)
        for j in range(3)]
    aux_cps += [
        pltpu.async_copy(hap_hbm, hap_v, sem_a),
        pltpu.async_copy(hbp_hbm, hbp_v, sem_a),
        pltpu.async_copy(ttab_hbm, t_v, sem_a),
        pltpu.async_copy(g_hbm, g_v.at[pl.ds(0, 12)], sem_a),
        pltpu.async_copy(b_hbm, b_v.at[pl.ds(0, 12)], sem_a),
    ]
    for cp in idx_cps:
        cp.wait()

    copies = []
    for t in range(3):
        for c in range(NCHUNK):
            sl = pl.ds(c * CHUNK, CHUNK)
            copies.append(pltpu.async_copy(
                tabs[t].at[c_v[3 + t].at[sl]], e_v[t][0].at[sl], sem))
    for t in range(3):
        for g in range(GROUPS):
            ib_v[t][pl.ds(16 * g, 16)] = c_v[3 + t][pl.ds(16 * g, 16)] + V
        for c in range(NCHUNK):
            sl = pl.ds(c * CHUNK, CHUNK)
            copies.append(pltpu.async_copy(
                tabs[t].at[ib_v[t].at[sl]], e_v[t][1].at[sl], sem))
    for cp in aux_cps:
        cp.wait()

    gvec = g_v[...]
    bvec = b_v[...]
    scale = [gvec[j] * INV for j in range(12)]
    bias = [bvec[j] for j in range(12)]

    def emit(col, t, ob):
        v = jnp.maximum(t * scale[col] + bias[col], 0.0)
        plsc.store_scatter(o_v, [ob + col], v)

    for g in range(GROUPS):
        ob = 192 * g + 12 * i16
        for j in range(3):
            xv = c_v[j][pl.ds(16 * g, 16)]
            pa = plsc.load_gather(hap_v, [lax.shift_right_logical(xv, 8)])
            pb = plsc.load_gather(hbp_v, [lax.bitwise_and(xv, 255)])
            s = pa + pb
            if j == 0:
                t = _OFF12 + 2 * lax.bitwise_and(s, 31)
            elif j == 1:
                t = _OFF7 + 2 * lax.bitwise_and(
                    lax.shift_right_logical(s, _F7), 15)
            else:
                t = _OFF24 + 2 * lax.shift_right_logical(s, _F24)
            emit(2 * j, plsc.load_gather(t_v, [t]), ob)
            emit(2 * j + 1, plsc.load_gather(t_v, [t + 1]), ob)

    for cp in copies:
        cp.wait()

    for g in range(GROUPS):
        ob = 192 * g + 12 * i16
        for t in range(3):
            for col in range(2):
                emit(6 + 2 * t + col, e_v[t][col][pl.ds(16 * g, 16)], ob)

    pltpu.sync_copy(o_v, out_hbm.at[pl.ds(base * 12, ROWS * 12)])


def kernel(x, emb3, emb4, emb5, gamma, beta):
    xt = x.astype(jnp.int32).T.reshape(-1)
    out = _encode(xt, jnp.asarray(_HAP), jnp.asarray(_HBP),
                  jnp.asarray(_TTAB), emb3.T.reshape(-1), emb4.T.reshape(-1),
                  emb5.T.reshape(-1), gamma, beta)
    return out.reshape(B, 12)


# speedup vs baseline: 6.5128x; 1.5902x over previous
import functools
import math

import numpy as np
import jax
import jax.numpy as jnp
from jax import lax
from jax.experimental import pallas as pl
from jax.experimental.pallas import tpu as pltpu
from jax.experimental.pallas import tpu_sc as plsc

B = 16384
NW = 32
ROWS = B // NW
GROUPS = ROWS // 16
NCHUNK = 4
CHUNK = ROWS // NCHUNK
INV = 1.0 / math.sqrt(1.0 + 1e-5)
HMAX = 392
V = 100001

_F7, _F24 = 5, 9


def _pack(v: np.ndarray) -> np.ndarray:
    return (v % 12) | ((v % 7) << _F7) | ((v % 24) << _F24)


_HAP = _pack(256 * np.arange(HMAX, dtype=np.int64)).astype(np.int32)
_HBP = _pack(np.arange(256, dtype=np.int64)).astype(np.int32)


def _sincos_ext(m: int, n: int, phase: int) -> np.ndarray:
    k = np.arange(n, dtype=np.float64) + phase
    ang = 2.0 * np.pi * k / m
    return np.stack([np.sin(ang), np.cos(ang)], axis=1).reshape(-1)


_T12 = _sincos_ext(12, 23, 11)
_T7 = _sincos_ext(7, 13, 0)
_T24 = _sincos_ext(24, 47, 0)
_OFF12, _OFF7, _OFF24 = 0, _T12.size, _T12.size + _T7.size
_TTAB = np.concatenate([_T12, _T7, _T24]).astype(np.float32)
_TTAB = np.pad(_TTAB, (0, (-_TTAB.size) % 8))
TTN = _TTAB.size

_mesh = plsc.VectorSubcoreMesh(core_axis_name="c", subcore_axis_name="s")


@functools.partial(
    pl.kernel,
    mesh=_mesh,
    compiler_params=pltpu.CompilerParams(
        needs_layout_passes=False, use_tc_tiling_on_sc=False),
    out_type=jax.ShapeDtypeStruct((B * 12,), jnp.float32),
    scratch_types=[
        [pltpu.VMEM((ROWS,), jnp.int32) for _ in range(6)],
        [pltpu.VMEM((ROWS,), jnp.int32) for _ in range(3)],
        pltpu.VMEM((HMAX,), jnp.int32),
        pltpu.VMEM((256,), jnp.int32),
        pltpu.VMEM((TTN,), jnp.float32),
        pltpu.VMEM((16,), jnp.float32),
        pltpu.VMEM((16,), jnp.float32),
        [[pltpu.VMEM((ROWS,), jnp.float32) for _ in range(2)]
         for _ in range(3)],
        [pltpu.VMEM((ROWS,), jnp.float32) for _ in range(12)],
        pltpu.SemaphoreType.DMA,
        pltpu.SemaphoreType.DMA,
        pltpu.SemaphoreType.DMA,
    ],
)
def _encode(xt_hbm, hap_hbm, hbp_hbm, ttab_hbm, f3_hbm, f4_hbm, f5_hbm,
            g_hbm, b_hbm, out_hbm,
            c_v, ib_v, hap_v, hbp_v, t_v, g_v, b_v, e_v, o_v,
            sem, sem_i, sem_a):
    wid = lax.axis_index("s") * 2 + lax.axis_index("c")
    base = wid * ROWS
    i16 = lax.iota(jnp.int32, 16)
    tabs = (f3_hbm, f4_hbm, f5_hbm)

    idx_cps = [pltpu.async_copy(
        xt_hbm.at[pl.ds((3 + t) * B + base, ROWS)], c_v[3 + t], sem_i)
        for t in range(3)]
    aux_cps = [pltpu.async_copy(
        xt_hbm.at[pl.ds(j * B + base, ROWS)], c_v[j], sem_a)
        for j in range(3)]
    aux_cps += [
        pltpu.async_copy(hap_hbm, hap_v, sem_a),
        pltpu.async_copy(hbp_hbm, hbp_v, sem_a),
        pltpu.async_copy(ttab_hbm, t_v, sem_a),
        pltpu.async_copy(g_hbm, g_v.at[pl.ds(0, 12)], sem_a),
        pltpu.async_copy(b_hbm, b_v.at[pl.ds(0, 12)], sem_a),
    ]
    for cp in idx_cps:
        cp.wait()

    copies = []
    for t in range(3):
        for c in range(NCHUNK):
            sl = pl.ds(c * CHUNK, CHUNK)
            copies.append(pltpu.async_copy(
                tabs[t].at[c_v[3 + t].at[sl]], e_v[t][0].at[sl], sem))
    for t in range(3):
        for g in range(GROUPS):
            ib_v[t][pl.ds(16 * g, 16)] = c_v[3 + t][pl.ds(16 * g, 16)] + V
        for c in range(NCHUNK):
            sl = pl.ds(c * CHUNK, CHUNK)
            copies.append(pltpu.async_copy(
                tabs[t].at[ib_v[t].at[sl]], e_v[t][1].at[sl], sem))
    for cp in aux_cps:
        cp.wait()

    gvec = g_v[...]
    bvec = b_v[...]
    scale = [gvec[j] * INV for j in range(12)]
    bias = [bvec[j] for j in range(12)]

    def emit(col, t, sl):
        o_v[col][sl] = jnp.maximum(t * scale[col] + bias[col], 0.0)

    for g in range(GROUPS):
        sl = pl.ds(16 * g, 16)
        for j in range(3):
            xv = c_v[j][sl]
            pa = plsc.load_gather(hap_v, [lax.shift_right_logical(xv, 8)])
            pb = plsc.load_gather(hbp_v, [lax.bitwise_and(xv, 255)])
            s = pa + pb
            if j == 0:
                t = _OFF12 + 2 * lax.bitwise_and(s, 31)
            elif j == 1:
                t = _OFF7 + 2 * lax.bitwise_and(
                    lax.shift_right_logical(s, _F7), 15)
            else:
                t = _OFF24 + 2 * lax.shift_right_logical(s, _F24)
            emit(2 * j, plsc.load_gather(t_v, [t]), sl)
            emit(2 * j + 1, plsc.load_gather(t_v, [t + 1]), sl)

    for cp in copies:
        cp.wait()

    for g in range(GROUPS):
        sl = pl.ds(16 * g, 16)
        for t in range(3):
            for col in range(2):
                emit(6 + 2 * t + col, e_v[t][col][sl], sl)

    out_cps = [pltpu.async_copy(
        o_v[j], out_hbm.at[pl.ds(j * B + base, ROWS)], sem_i)
        for j in range(12)]
    for cp in out_cps:
        cp.wait()


def kernel(x, emb3, emb4, emb5, gamma, beta):
    xt = x.astype(jnp.int32).T.reshape(-1)
    out = _encode(xt, jnp.asarray(_HAP), jnp.asarray(_HBP),
                  jnp.asarray(_TTAB), emb3.T.reshape(-1), emb4.T.reshape(-1),
                  emb5.T.reshape(-1), gamma, beta)
    return out.reshape(12, B).T
